# Initial kernel scaffold; baseline (speedup 1.0000x reference)
import jax
import jax.numpy as jnp
from jax.experimental import pallas as pl


def kernel(in_slew, load, axis_0, axis_1, delay_table, slew_table):
    raise NotImplementedError("write your pallas kernel here")


# speedup vs baseline: 1391.2717x; 1391.2717x over previous
import functools

import jax
import jax.numpy as jnp
from jax import lax
from jax.experimental import pallas as pl
from jax.experimental.pallas import tpu as pltpu
from jax.experimental.pallas import tpu_sc as plsc

_NC = 2
_NS = 16
_NW = _NC * _NS
_L = 16
_CH = 8192


def _sc_body(x_hbm, y_hbm, thr_hbm, ax_hbm, ct_hbm, dly_hbm, slw_hbm,
             xv, yv, dv, sv, thrv, axv, ctv):
    per_w = x_hbm.shape[0] // _NW
    n_ch = per_w // _CH
    wid = lax.axis_index("s") * _NC + lax.axis_index("c")
    base = wid * per_w

    pltpu.sync_copy(thr_hbm, thrv)
    pltpu.sync_copy(ax_hbm, axv)
    pltpu.sync_copy(ct_hbm, ctv)

    t0 = [thrv[k] for k in range(7)]
    t1 = [thrv[8 + k] for k in range(7)]
    kconst = [jnp.full((_L,), k, jnp.int32) for k in range(1, 8)]
    six = jnp.full((_L,), 6, jnp.int32)

    def chunk(ci, carry):
        off = base + ci * _CH
        pltpu.sync_copy(x_hbm.at[pl.ds(off, _CH)], xv)
        pltpu.sync_copy(y_hbm.at[pl.ds(off, _CH)], yv)

        def vec(vi, carry2):
            o = vi * _L
            x = xv[pl.ds(o, _L)]
            y = yv[pl.ds(o, _L)]
            i0 = jnp.zeros((_L,), jnp.int32)
            j0 = jnp.zeros((_L,), jnp.int32)
            for k in range(7):
                i0 = jnp.where(x >= t0[k], kconst[k], i0)
                j0 = jnp.where(y >= t1[k], kconst[k], j0)
            i0 = jnp.minimum(i0, six)
            j0 = jnp.minimum(j0, six)
            x0 = plsc.load_gather(axv, [i0])
            r0 = plsc.load_gather(axv, [i0 + 8])
            y0 = plsc.load_gather(axv, [j0 + 16])
            r1 = plsc.load_gather(axv, [j0 + 24])
            a = (x - x0) * r0
            b = (y - y0) * r1
            ab = a * b
            cell = i0 * 8 + j0
            d = plsc.load_gather(ctv, [cell])
            d = d + a * plsc.load_gather(ctv, [cell + 64])
            d = d + b * plsc.load_gather(ctv, [cell + 128])
            d = d + ab * plsc.load_gather(ctv, [cell + 192])
            s = plsc.load_gather(ctv, [cell + 256])
            s = s + a * plsc.load_gather(ctv, [cell + 320])
            s = s + b * plsc.load_gather(ctv, [cell + 384])
            s = s + ab * plsc.load_gather(ctv, [cell + 448])
            dv[pl.ds(o, _L)] = d
            sv[pl.ds(o, _L)] = s
            return carry2

        lax.fori_loop(0, _CH // _L, vec, 0)
        pltpu.sync_copy(dv, dly_hbm.at[pl.ds(off, _CH)])
        pltpu.sync_copy(sv, slw_hbm.at[pl.ds(off, _CH)])
        return carry

    lax.fori_loop(0, n_ch, chunk, 0)


def _coef_tables(tab):
    c0 = tab[:-1, :-1]
    ca = tab[1:, :-1] - tab[:-1, :-1]
    cb = tab[:-1, 1:] - tab[:-1, :-1]
    cab = tab[1:, 1:] - tab[1:, :-1] - tab[:-1, 1:] + tab[:-1, :-1]
    pad = lambda c: jnp.pad(c, ((0, 1), (0, 1))).reshape(-1)
    return [pad(c) for c in (c0, ca, cb, cab)]


def kernel(in_slew, load, axis_0, axis_1, delay_table, slew_table):
    M = in_slew.shape[0]
    eps = 1e-30

    def recip(axis):
        d = axis[1:] - axis[:-1]
        r = jnp.where(jnp.abs(d) > eps, 1.0 / (d + eps), jnp.zeros_like(d))
        return jnp.pad(r, (0, 1)).astype(jnp.float32)

    ax = jnp.concatenate(
        [axis_0, recip(axis_0), axis_1, recip(axis_1),
         jnp.zeros((96,), jnp.float32)]).astype(jnp.float32)
    ct = jnp.concatenate(
        _coef_tables(delay_table) + _coef_tables(slew_table)).astype(jnp.float32)
    thr = jnp.zeros((16, _L), jnp.float32)
    thr = thr.at[0:7].set(axis_0[1:8, None])
    thr = thr.at[8:15].set(axis_1[1:8, None])

    blk = _NW * _CH
    Mp = -(-M // blk) * blk
    x = in_slew.astype(jnp.float32)
    y = load.astype(jnp.float32)
    if Mp != M:
        x = jnp.pad(x, (0, Mp - M))
        y = jnp.pad(y, (0, Mp - M))

    mesh = plsc.VectorSubcoreMesh(core_axis_name="c", subcore_axis_name="s")
    out = jax.ShapeDtypeStruct((Mp,), jnp.float32)
    kfn = pl.kernel(
        _sc_body,
        mesh=mesh,
        out_type=(out, out),
        compiler_params=pltpu.CompilerParams(needs_layout_passes=False),
        scratch_types=[
            pltpu.VMEM((_CH,), jnp.float32),
            pltpu.VMEM((_CH,), jnp.float32),
            pltpu.VMEM((_CH,), jnp.float32),
            pltpu.VMEM((_CH,), jnp.float32),
            pltpu.VMEM((16, _L), jnp.float32),
            pltpu.VMEM((128,), jnp.float32),
            pltpu.VMEM((512,), jnp.float32),
        ],
    )
    delay, slew = kfn(x, y, thr, ax, ct)
    if Mp != M:
        delay = delay[:M]
        slew = slew[:M]
    return delay, slew
